# baseline (speedup 1.0000x reference)
import functools

import jax
import jax.numpy as jnp
from jax import lax
from jax.experimental import pallas as pl
from jax.experimental.pallas import tpu as pltpu
from jax.experimental.pallas import tpu_sc as plsc

_NBUF = 2
_TBLK = 512


def _make_widen(v: int, d: int):
    grid = (v + _TBLK - 1) // _TBLK

    def widen(in_ref, out_ref):
        x = in_ref[...]
        eye = (lax.broadcasted_iota(jnp.int32, (d, d), 0) ==
               lax.broadcasted_iota(jnp.int32, (d, d), 1)).astype(jnp.float32)
        out_ref[:, 0:d] = lax.dot_general(
            x, eye, (((0,), (0,)), ((), ())),
            precision=lax.Precision.HIGHEST,
            preferred_element_type=jnp.float32)

    return pl.pallas_call(
        widen,
        grid=(grid,),
        in_specs=[pl.BlockSpec((d, _TBLK), lambda j: (0, j))],
        out_specs=pl.BlockSpec((_TBLK, 2 * d), lambda j: (j, 0)),
        out_shape=jax.ShapeDtypeStruct((v, 2 * d), jnp.float32),
    )


def _make_gather(n_workers: int, per_w: int, chunk: int, n_ch: int,
                 n_total: int, d: int):
    mesh = plsc.VectorSubcoreMesh(core_axis_name="c", subcore_axis_name="s")

    @functools.partial(
        pl.kernel,
        mesh=mesh,
        out_type=jax.ShapeDtypeStruct((n_total, d), jnp.float32),
        scratch_types=[
            pltpu.VMEM((n_ch, chunk), jnp.int32),
            pltpu.VMEM((_NBUF, chunk, 2 * d), jnp.float32),
            pltpu.VMEM((_NBUF, chunk, d), jnp.float32),
            pltpu.SemaphoreType.DMA,
            pltpu.SemaphoreType.DMA,
        ],
        compiler_params=pltpu.CompilerParams(
            use_tc_tiling_on_sc=True, needs_layout_passes=False),
    )
    def gather_kernel(table_hbm, idx_hbm, out_hbm, idx_v, bufs, obufs,
                      gsem, osem):
        wid = lax.axis_index("s") * 2 + lax.axis_index("c")
        base = wid * per_w
        pltpu.sync_copy(idx_hbm.at[wid], idx_v)

        n_outer = n_ch // _NBUF
        n_lane = 16

        def compact(buf, obuf):
            for k in range(chunk):
                for q in range(d // n_lane):
                    obuf[k, pl.ds(q * n_lane, n_lane)] = (
                        buf[k, pl.ds(q * n_lane, n_lane)])

        def fire_gather(j, b):
            return pltpu.async_copy(table_hbm.at[idx_v.at[j]],
                                    bufs.at[b], gsem)

        def wait_out(j, b):
            pltpu.make_async_copy(
                obufs.at[b],
                out_hbm.at[pl.ds(base + j * chunk, chunk)], osem).wait()

        for b in range(_NBUF):
            fire_gather(b, b)

        def body(jj, carry):
            j0 = jj * _NBUF
            for b in range(_NBUF):
                j = j0 + b
                pltpu.make_async_copy(table_hbm.at[idx_v.at[j]],
                                      bufs.at[b], gsem).wait()

                @pl.when(j >= _NBUF)
                def _():
                    wait_out(j - _NBUF, b)

                compact(bufs.at[b], obufs.at[b])

                @pl.when(j + _NBUF < n_ch)
                def _():
                    fire_gather(j + _NBUF, b)

                dst = out_hbm.at[pl.ds(base + j * chunk, chunk)]
                pltpu.async_copy(obufs.at[b], dst, osem)
            return carry

        lax.fori_loop(0, n_outer, body, 0)
        for b in range(_NBUF):
            wait_out(n_ch - _NBUF + b, b)

    return gather_kernel


def kernel(trg_seq, enc_output, W):
    del enc_output
    batch, seq = trg_seq.shape
    v, d = W.shape
    n_total = batch * seq

    n_workers = 32
    per_w = n_total // n_workers
    chunk = 128
    n_ch = per_w // chunk

    w128 = _make_widen(v, d)(W.T)
    idx = trg_seq.reshape(n_workers, n_ch, chunk).astype(jnp.int32)
    fn = _make_gather(n_workers, per_w, chunk, n_ch, n_total, d)
    out = fn(w128, idx)
    return out.reshape(batch, seq, d)


# speedup vs baseline: 2.0005x; 2.0005x over previous
import functools

import jax
import jax.numpy as jnp
from jax import lax
from jax.experimental import pallas as pl
from jax.experimental.pallas import tpu as pltpu
from jax.experimental.pallas import tpu_sc as plsc

_NBUF = 2
_TBLK = 2048


def _make_widen(v: int, d: int):
    grid = (v + _TBLK - 1) // _TBLK

    def widen(in_ref, out_ref):
        out_ref[:, 0:d] = in_ref[...].T

    return pl.pallas_call(
        widen,
        grid=(grid,),
        in_specs=[pl.BlockSpec((d, _TBLK), lambda j: (0, j))],
        out_specs=pl.BlockSpec((_TBLK, 2 * d), lambda j: (j, 0)),
        out_shape=jax.ShapeDtypeStruct((v, 2 * d), jnp.float32),
    )


def _make_gather(n_workers: int, per_w: int, chunk: int, n_ch: int,
                 n_total: int, d: int):
    mesh = plsc.VectorSubcoreMesh(core_axis_name="c", subcore_axis_name="s")

    @functools.partial(
        pl.kernel,
        mesh=mesh,
        out_type=jax.ShapeDtypeStruct((n_total, d), jnp.float32),
        scratch_types=[
            pltpu.VMEM((n_ch, chunk), jnp.int32),
            pltpu.VMEM((_NBUF, chunk, 2 * d), jnp.float32),
            pltpu.VMEM((_NBUF, chunk, d), jnp.float32),
            pltpu.SemaphoreType.DMA,
            pltpu.SemaphoreType.DMA,
        ],
        compiler_params=pltpu.CompilerParams(
            use_tc_tiling_on_sc=True, needs_layout_passes=False),
    )
    def gather_kernel(table_hbm, idx_hbm, out_hbm, idx_v, bufs, obufs,
                      gsem, osem):
        wid = lax.axis_index("s") * 2 + lax.axis_index("c")
        base = wid * per_w
        pltpu.sync_copy(idx_hbm.at[wid], idx_v)

        n_outer = n_ch // _NBUF
        n_lane = 16

        def compact(buf, obuf):
            for k in range(chunk):
                for q in range(d // n_lane):
                    obuf[k, pl.ds(q * n_lane, n_lane)] = (
                        buf[k, pl.ds(q * n_lane, n_lane)])

        def fire_gather(j, b):
            return pltpu.async_copy(table_hbm.at[idx_v.at[j]],
                                    bufs.at[b], gsem)

        def wait_out(j, b):
            pltpu.make_async_copy(
                obufs.at[b],
                out_hbm.at[pl.ds(base + j * chunk, chunk)], osem).wait()

        for b in range(_NBUF):
            fire_gather(b, b)

        def body(jj, carry):
            j0 = jj * _NBUF
            for b in range(_NBUF):
                j = j0 + b
                pltpu.make_async_copy(table_hbm.at[idx_v.at[j]],
                                      bufs.at[b], gsem).wait()

                @pl.when(j >= _NBUF)
                def _():
                    wait_out(j - _NBUF, b)

                compact(bufs.at[b], obufs.at[b])

                @pl.when(j + _NBUF < n_ch)
                def _():
                    fire_gather(j + _NBUF, b)

                dst = out_hbm.at[pl.ds(base + j * chunk, chunk)]
                pltpu.async_copy(obufs.at[b], dst, osem)
            return carry

        lax.fori_loop(0, n_outer, body, 0)
        for b in range(_NBUF):
            wait_out(n_ch - _NBUF + b, b)

    return gather_kernel


def kernel(trg_seq, enc_output, W):
    del enc_output
    batch, seq = trg_seq.shape
    v, d = W.shape
    n_total = batch * seq

    n_workers = 32
    per_w = n_total // n_workers
    chunk = 128
    n_ch = per_w // chunk

    w128 = _make_widen(v, d)(W.T)
    idx = trg_seq.reshape(n_workers, n_ch, chunk).astype(jnp.int32)
    fn = _make_gather(n_workers, per_w, chunk, n_ch, n_total, d)
    out = fn(w128, idx)
    return out.reshape(batch, seq, d)


# speedup vs baseline: 2.2963x; 1.1479x over previous
import functools

import jax
import jax.numpy as jnp
from jax import lax
from jax.experimental import pallas as pl
from jax.experimental.pallas import tpu as pltpu
from jax.experimental.pallas import tpu_sc as plsc

_NBUF = 2
_TBLK = 4096


def _make_widen(v: int, d: int):
    grid = (v + _TBLK - 1) // _TBLK

    def widen(in_ref, out_ref):
        out_ref[:, 0:d] = in_ref[...].T

    return pl.pallas_call(
        widen,
        grid=(grid,),
        in_specs=[pl.BlockSpec((d, _TBLK), lambda j: (0, j))],
        out_specs=pl.BlockSpec((_TBLK, 2 * d), lambda j: (j, 0)),
        out_shape=jax.ShapeDtypeStruct((v, 2 * d), jnp.float32),
    )


def _make_gather(n_workers: int, per_w: int, chunk: int, n_ch: int,
                 n_total: int, d: int):
    mesh = plsc.VectorSubcoreMesh(core_axis_name="c", subcore_axis_name="s")

    @functools.partial(
        pl.kernel,
        mesh=mesh,
        out_type=jax.ShapeDtypeStruct((n_total, d), jnp.float32),
        scratch_types=[
            pltpu.VMEM((n_ch, chunk), jnp.int32),
            pltpu.VMEM((_NBUF, chunk, 2 * d), jnp.float32),
            pltpu.VMEM((_NBUF, chunk, d), jnp.float32),
            pltpu.SemaphoreType.DMA,
            pltpu.SemaphoreType.DMA,
        ],
        compiler_params=pltpu.CompilerParams(
            use_tc_tiling_on_sc=True, needs_layout_passes=False),
    )
    def gather_kernel(table_hbm, idx_hbm, out_hbm, idx_v, bufs, obufs,
                      gsem, osem):
        wid = lax.axis_index("s") * 2 + lax.axis_index("c")
        base = wid * per_w
        pltpu.sync_copy(idx_hbm.at[wid], idx_v)

        n_outer = n_ch // _NBUF
        n_lane = 16

        def compact(buf, obuf):
            for k in range(chunk):
                for q in range(d // n_lane):
                    obuf[k, pl.ds(q * n_lane, n_lane)] = (
                        buf[k, pl.ds(q * n_lane, n_lane)])

        def fire_gather(j, b):
            return pltpu.async_copy(table_hbm.at[idx_v.at[j]],
                                    bufs.at[b], gsem)

        def wait_out(j, b):
            pltpu.make_async_copy(
                obufs.at[b],
                out_hbm.at[pl.ds(base + j * chunk, chunk)], osem).wait()

        for b in range(_NBUF):
            fire_gather(b, b)

        def body(jj, carry):
            j0 = jj * _NBUF
            for b in range(_NBUF):
                j = j0 + b
                pltpu.make_async_copy(table_hbm.at[idx_v.at[j]],
                                      bufs.at[b], gsem).wait()

                @pl.when(j >= _NBUF)
                def _():
                    wait_out(j - _NBUF, b)

                compact(bufs.at[b], obufs.at[b])

                @pl.when(j + _NBUF < n_ch)
                def _():
                    fire_gather(j + _NBUF, b)

                dst = out_hbm.at[pl.ds(base + j * chunk, chunk)]
                pltpu.async_copy(obufs.at[b], dst, osem)
            return carry

        lax.fori_loop(0, n_outer, body, 0)
        for b in range(_NBUF):
            wait_out(n_ch - _NBUF + b, b)

    return gather_kernel


def kernel(trg_seq, enc_output, W):
    del enc_output
    batch, seq = trg_seq.shape
    v, d = W.shape
    n_total = batch * seq

    n_workers = 32
    per_w = n_total // n_workers
    chunk = 128
    n_ch = per_w // chunk

    w128 = _make_widen(v, d)(W.T)
    idx = trg_seq.reshape(n_workers, n_ch, chunk).astype(jnp.int32)
    fn = _make_gather(n_workers, per_w, chunk, n_ch, n_total, d)
    out = fn(w128, idx)
    return out.reshape(batch, seq, d)


# speedup vs baseline: 2.5221x; 1.0983x over previous
import functools

import jax
import jax.numpy as jnp
from jax import lax
from jax.experimental import pallas as pl
from jax.experimental.pallas import tpu as pltpu
from jax.experimental.pallas import tpu_sc as plsc

_NBUF = 2
_TBLK = 8192


def _make_widen(v: int, d: int):
    grid = (v + _TBLK - 1) // _TBLK

    def widen(in_ref, out_ref):
        out_ref[:, 0:d] = in_ref[...].T

    return pl.pallas_call(
        widen,
        grid=(grid,),
        in_specs=[pl.BlockSpec((d, _TBLK), lambda j: (0, j))],
        out_specs=pl.BlockSpec((_TBLK, 2 * d), lambda j: (j, 0)),
        out_shape=jax.ShapeDtypeStruct((v, 2 * d), jnp.float32),
    )


def _make_gather(n_workers: int, per_w: int, chunk: int, n_ch: int,
                 n_total: int, d: int):
    mesh = plsc.VectorSubcoreMesh(core_axis_name="c", subcore_axis_name="s")

    @functools.partial(
        pl.kernel,
        mesh=mesh,
        out_type=jax.ShapeDtypeStruct((n_total, d), jnp.float32),
        scratch_types=[
            pltpu.VMEM((n_ch, chunk), jnp.int32),
            pltpu.VMEM((_NBUF, chunk, 2 * d), jnp.float32),
            pltpu.VMEM((_NBUF, chunk, d), jnp.float32),
            pltpu.SemaphoreType.DMA,
            pltpu.SemaphoreType.DMA,
        ],
        compiler_params=pltpu.CompilerParams(
            use_tc_tiling_on_sc=True, needs_layout_passes=False),
    )
    def gather_kernel(table_hbm, idx_hbm, out_hbm, idx_v, bufs, obufs,
                      gsem, osem):
        wid = lax.axis_index("s") * 2 + lax.axis_index("c")
        base = wid * per_w
        pltpu.sync_copy(idx_hbm.at[wid], idx_v)

        n_outer = n_ch // _NBUF
        n_lane = 16

        def compact(buf, obuf):
            for k in range(chunk):
                for q in range(d // n_lane):
                    obuf[k, pl.ds(q * n_lane, n_lane)] = (
                        buf[k, pl.ds(q * n_lane, n_lane)])

        def fire_gather(j, b):
            return pltpu.async_copy(table_hbm.at[idx_v.at[j]],
                                    bufs.at[b], gsem)

        def wait_out(j, b):
            pltpu.make_async_copy(
                obufs.at[b],
                out_hbm.at[pl.ds(base + j * chunk, chunk)], osem).wait()

        for b in range(_NBUF):
            fire_gather(b, b)

        def body(jj, carry):
            j0 = jj * _NBUF
            for b in range(_NBUF):
                j = j0 + b
                pltpu.make_async_copy(table_hbm.at[idx_v.at[j]],
                                      bufs.at[b], gsem).wait()

                @pl.when(j >= _NBUF)
                def _():
                    wait_out(j - _NBUF, b)

                compact(bufs.at[b], obufs.at[b])

                @pl.when(j + _NBUF < n_ch)
                def _():
                    fire_gather(j + _NBUF, b)

                dst = out_hbm.at[pl.ds(base + j * chunk, chunk)]
                pltpu.async_copy(obufs.at[b], dst, osem)
            return carry

        lax.fori_loop(0, n_outer, body, 0)
        for b in range(_NBUF):
            wait_out(n_ch - _NBUF + b, b)

    return gather_kernel


def kernel(trg_seq, enc_output, W):
    del enc_output
    batch, seq = trg_seq.shape
    v, d = W.shape
    n_total = batch * seq

    n_workers = 32
    per_w = n_total // n_workers
    chunk = 128
    n_ch = per_w // chunk

    w128 = _make_widen(v, d)(W.T)
    idx = trg_seq.reshape(n_workers, n_ch, chunk).astype(jnp.int32)
    fn = _make_gather(n_workers, per_w, chunk, n_ch, n_total, d)
    out = fn(w128, idx)
    return out.reshape(batch, seq, d)


# speedup vs baseline: 2.5823x; 1.0239x over previous
import functools

import jax
import jax.numpy as jnp
from jax import lax
from jax.experimental import pallas as pl
from jax.experimental.pallas import tpu as pltpu
from jax.experimental.pallas import tpu_sc as plsc

_NBUF = 2
_TBLK = 16384


def _make_widen(v: int, d: int):
    grid = (v + _TBLK - 1) // _TBLK

    def widen(in_ref, out_ref):
        out_ref[:, 0:d] = in_ref[...].T

    return pl.pallas_call(
        widen,
        grid=(grid,),
        in_specs=[pl.BlockSpec((d, _TBLK), lambda j: (0, j))],
        out_specs=pl.BlockSpec((_TBLK, 2 * d), lambda j: (j, 0)),
        out_shape=jax.ShapeDtypeStruct((v, 2 * d), jnp.float32),
    )


def _make_gather(n_workers: int, per_w: int, chunk: int, n_ch: int,
                 n_total: int, d: int):
    mesh = plsc.VectorSubcoreMesh(core_axis_name="c", subcore_axis_name="s")

    @functools.partial(
        pl.kernel,
        mesh=mesh,
        out_type=jax.ShapeDtypeStruct((n_total, d), jnp.float32),
        scratch_types=[
            pltpu.VMEM((n_ch, chunk), jnp.int32),
            pltpu.VMEM((_NBUF, chunk, 2 * d), jnp.float32),
            pltpu.VMEM((_NBUF, chunk, d), jnp.float32),
            pltpu.SemaphoreType.DMA,
            pltpu.SemaphoreType.DMA,
        ],
        compiler_params=pltpu.CompilerParams(
            use_tc_tiling_on_sc=True, needs_layout_passes=False),
    )
    def gather_kernel(table_hbm, idx_hbm, out_hbm, idx_v, bufs, obufs,
                      gsem, osem):
        wid = lax.axis_index("s") * 2 + lax.axis_index("c")
        base = wid * per_w
        pltpu.sync_copy(idx_hbm.at[wid], idx_v)

        n_outer = n_ch // _NBUF
        n_lane = 16

        def compact(buf, obuf):
            for k in range(chunk):
                for q in range(d // n_lane):
                    obuf[k, pl.ds(q * n_lane, n_lane)] = (
                        buf[k, pl.ds(q * n_lane, n_lane)])

        def fire_gather(j, b):
            return pltpu.async_copy(table_hbm.at[idx_v.at[j]],
                                    bufs.at[b], gsem)

        def wait_out(j, b):
            pltpu.make_async_copy(
                obufs.at[b],
                out_hbm.at[pl.ds(base + j * chunk, chunk)], osem).wait()

        for b in range(_NBUF):
            fire_gather(b, b)

        def body(jj, carry):
            j0 = jj * _NBUF
            for b in range(_NBUF):
                j = j0 + b
                pltpu.make_async_copy(table_hbm.at[idx_v.at[j]],
                                      bufs.at[b], gsem).wait()

                @pl.when(j >= _NBUF)
                def _():
                    wait_out(j - _NBUF, b)

                compact(bufs.at[b], obufs.at[b])

                @pl.when(j + _NBUF < n_ch)
                def _():
                    fire_gather(j + _NBUF, b)

                dst = out_hbm.at[pl.ds(base + j * chunk, chunk)]
                pltpu.async_copy(obufs.at[b], dst, osem)
            return carry

        lax.fori_loop(0, n_outer, body, 0)
        for b in range(_NBUF):
            wait_out(n_ch - _NBUF + b, b)

    return gather_kernel


def kernel(trg_seq, enc_output, W):
    del enc_output
    batch, seq = trg_seq.shape
    v, d = W.shape
    n_total = batch * seq

    n_workers = 32
    per_w = n_total // n_workers
    chunk = 128
    n_ch = per_w // chunk

    w128 = _make_widen(v, d)(W.T)
    idx = trg_seq.reshape(n_workers, n_ch, chunk).astype(jnp.int32)
    fn = _make_gather(n_workers, per_w, chunk, n_ch, n_total, d)
    out = fn(w128, idx)
    return out.reshape(batch, seq, d)


# speedup vs baseline: 2.6063x; 1.0093x over previous
import functools

import jax
import jax.numpy as jnp
from jax import lax
from jax.experimental import pallas as pl
from jax.experimental.pallas import tpu as pltpu
from jax.experimental.pallas import tpu_sc as plsc

_NBUF = 2
_TBLK = 32768


def _make_widen(v: int, d: int):
    grid = (v + _TBLK - 1) // _TBLK

    def widen(in_ref, out_ref):
        out_ref[:, 0:d] = in_ref[...].T

    return pl.pallas_call(
        widen,
        grid=(grid,),
        in_specs=[pl.BlockSpec((d, _TBLK), lambda j: (0, j))],
        out_specs=pl.BlockSpec((_TBLK, 2 * d), lambda j: (j, 0)),
        out_shape=jax.ShapeDtypeStruct((v, 2 * d), jnp.float32),
    )


def _make_gather(n_workers: int, per_w: int, chunk: int, n_ch: int,
                 n_total: int, d: int):
    mesh = plsc.VectorSubcoreMesh(core_axis_name="c", subcore_axis_name="s")

    @functools.partial(
        pl.kernel,
        mesh=mesh,
        out_type=jax.ShapeDtypeStruct((n_total, d), jnp.float32),
        scratch_types=[
            pltpu.VMEM((n_ch, chunk), jnp.int32),
            pltpu.VMEM((_NBUF, chunk, 2 * d), jnp.float32),
            pltpu.VMEM((_NBUF, chunk, d), jnp.float32),
            pltpu.SemaphoreType.DMA,
            pltpu.SemaphoreType.DMA,
        ],
        compiler_params=pltpu.CompilerParams(
            use_tc_tiling_on_sc=True, needs_layout_passes=False),
    )
    def gather_kernel(table_hbm, idx_hbm, out_hbm, idx_v, bufs, obufs,
                      gsem, osem):
        wid = lax.axis_index("s") * 2 + lax.axis_index("c")
        base = wid * per_w
        pltpu.sync_copy(idx_hbm.at[wid], idx_v)

        n_outer = n_ch // _NBUF
        n_lane = 16

        def compact(buf, obuf):
            for k in range(chunk):
                for q in range(d // n_lane):
                    obuf[k, pl.ds(q * n_lane, n_lane)] = (
                        buf[k, pl.ds(q * n_lane, n_lane)])

        def fire_gather(j, b):
            return pltpu.async_copy(table_hbm.at[idx_v.at[j]],
                                    bufs.at[b], gsem)

        def wait_out(j, b):
            pltpu.make_async_copy(
                obufs.at[b],
                out_hbm.at[pl.ds(base + j * chunk, chunk)], osem).wait()

        for b in range(_NBUF):
            fire_gather(b, b)

        def body(jj, carry):
            j0 = jj * _NBUF
            for b in range(_NBUF):
                j = j0 + b
                pltpu.make_async_copy(table_hbm.at[idx_v.at[j]],
                                      bufs.at[b], gsem).wait()

                @pl.when(j >= _NBUF)
                def _():
                    wait_out(j - _NBUF, b)

                compact(bufs.at[b], obufs.at[b])

                @pl.when(j + _NBUF < n_ch)
                def _():
                    fire_gather(j + _NBUF, b)

                dst = out_hbm.at[pl.ds(base + j * chunk, chunk)]
                pltpu.async_copy(obufs.at[b], dst, osem)
            return carry

        lax.fori_loop(0, n_outer, body, 0)
        for b in range(_NBUF):
            wait_out(n_ch - _NBUF + b, b)

    return gather_kernel


def kernel(trg_seq, enc_output, W):
    del enc_output
    batch, seq = trg_seq.shape
    v, d = W.shape
    n_total = batch * seq

    n_workers = 32
    per_w = n_total // n_workers
    chunk = 128
    n_ch = per_w // chunk

    w128 = _make_widen(v, d)(W.T)
    idx = trg_seq.reshape(n_workers, n_ch, chunk).astype(jnp.int32)
    fn = _make_gather(n_workers, per_w, chunk, n_ch, n_total, d)
    out = fn(w128, idx)
    return out.reshape(batch, seq, d)
